# baseline (speedup 1.0000x reference)
import functools

import jax
import jax.numpy as jnp
from jax import lax
from jax.experimental import pallas as pl
from jax.experimental.pallas import tpu as pltpu
from jax.experimental.pallas import tpu_sc as plsc

EMB_DIM = 64
BCHUNK = 128


@functools.cache
def _make_gather(batch: int, n_fields: int, n_pairs: int):
  NC, NS = 2, 16
  NW = NC * NS
  assert batch % BCHUNK == 0
  blocks_per_field = batch // BCHUNK
  n_chunks = n_fields * blocks_per_field
  assert n_chunks % NW == 0
  ch_per_w = n_chunks // NW
  assert ch_per_w % 4 == 0

  mesh = plsc.VectorSubcoreMesh(core_axis_name="c", subcore_axis_name="s")

  @functools.partial(
      pl.kernel,
      out_type=jax.ShapeDtypeStruct((n_fields, EMB_DIM, batch), jnp.float32),
      mesh=mesh,
      scratch_types=[
          pltpu.VMEM((ch_per_w, BCHUNK), jnp.int32),
          pltpu.VMEM((ch_per_w, BCHUNK), jnp.int32),
          pltpu.VMEM((4, BCHUNK, 2 * EMB_DIM), jnp.float32),
          pltpu.VMEM((2, EMB_DIM, BCHUNK + 1), jnp.float32),
          pltpu.SemaphoreType.DMA((4,)),
          pltpu.SemaphoreType.DMA((2,)),
      ],
      compiler_params=pltpu.CompilerParams(
          use_tc_tiling_on_sc=False, needs_layout_passes=False
      ),
  )
  def grab(idx_hbm, table_hbm, out_hbm, idx_v, sub_v, rows_v, obuf_v,
           gsem, osem):
    wid = lax.axis_index("s") * NC + lax.axis_index("c")
    base_chunk = wid * ch_per_w
    pltpu.sync_copy(idx_hbm.at[pl.ds(base_chunk, ch_per_w)], idx_v)

    @plsc.parallel_loop(0, ch_per_w * (BCHUNK // 16), unroll=4)
    def _(t):
      c = t // (BCHUNK // 16)
      s = (t % (BCHUNK // 16)) * 16
      v = idx_v[c, pl.ds(s, 16)]
      sub_v[c, pl.ds(s, 16)] = (v & 1) << 6
      idx_v[c, pl.ds(s, 16)] = v >> 1

    def gather(c, p):
      return pltpu.make_async_copy(
          table_hbm.at[idx_v.at[c]], rows_v.at[p], gsem.at[p]
      )

    def store(c, q):
      ci = base_chunk + c
      f = ci // blocks_per_field
      b0 = (ci % blocks_per_field) * BCHUNK
      return pltpu.make_async_copy(
          obuf_v.at[q, :, pl.ds(0, BCHUNK)],
          out_hbm.at[f, :, pl.ds(b0, BCHUNK)],
          osem.at[q],
      )

    iota16 = lax.iota(jnp.int32, 16)
    jvecs = [iota16 + 16 * g for g in range(EMB_DIM // 16)]

    gather(0, 0).start()
    gather(1, 1).start()

    @pl.loop(0, ch_per_w, step=4)
    def _(c0):
      for p in range(4):
        c = c0 + p
        q = p % 2
        gather(c, p).wait()

        @pl.when(c + 2 < ch_per_w)
        def _():
          gather(c + 2, (p + 2) % 4).start()

        @pl.when(c >= 2)
        def _():
          store(c - 2, q).wait()

        src = rows_v.at[p]
        dst = obuf_v.at[q]
        cc = jnp.full((16,), c, jnp.int32)

        @plsc.parallel_loop(0, BCHUNK, unroll=4)
        def _(b):
          bb = jnp.full((16,), b, jnp.int32)
          sv = plsc.load_gather(sub_v, [cc, bb])
          for g in range(EMB_DIM // 16):
            v = plsc.load_gather(src, [bb, sv + jvecs[g]])
            plsc.store_scatter(dst, [jvecs[g], bb], v)

        store(c, q).start()

    store(ch_per_w - 2, 0).wait()
    store(ch_per_w - 1, 1).wait()

  return grab


def kernel(input, embedding_weight):
  b, f = input.shape
  n_emb, d = embedding_weight.shape
  idx = input.astype(jnp.int32).T.reshape(f * (b // BCHUNK), BCHUNK)
  table2 = embedding_weight.reshape(n_emb // 2, 2 * d)
  grab = _make_gather(b, f, n_emb // 2)
  out_t = grab(idx, table2)
  return out_t.transpose(2, 0, 1).astype(jnp.bfloat16)


# speedup vs baseline: 1.0505x; 1.0505x over previous
import functools

import jax
import jax.numpy as jnp
from jax import lax
from jax.experimental import pallas as pl
from jax.experimental.pallas import tpu as pltpu
from jax.experimental.pallas import tpu_sc as plsc

EMB_DIM = 64
BCHUNK = 128


@functools.cache
def _make_gather(batch: int, n_fields: int, n_emb: int):
  NC, NS = 2, 16
  NW = NC * NS
  assert batch % BCHUNK == 0
  blocks_per_field = batch // BCHUNK
  n_chunks = n_fields * blocks_per_field
  assert n_chunks % NW == 0
  ch_per_w = n_chunks // NW
  assert ch_per_w % 4 == 0

  mesh = plsc.VectorSubcoreMesh(core_axis_name="c", subcore_axis_name="s")

  @functools.partial(
      pl.kernel,
      out_type=jax.ShapeDtypeStruct((n_fields, EMB_DIM, batch), jnp.float32),
      mesh=mesh,
      scratch_types=[
          pltpu.VMEM((ch_per_w, BCHUNK), jnp.int32),
          pltpu.VMEM((4, BCHUNK, EMB_DIM), jnp.float32),
          pltpu.VMEM((2, EMB_DIM, BCHUNK + 1), jnp.float32),
          pltpu.SemaphoreType.DMA((4,)),
          pltpu.SemaphoreType.DMA((2,)),
      ],
      compiler_params=pltpu.CompilerParams(
          use_tc_tiling_on_sc=False, needs_layout_passes=False
      ),
  )
  def grab(idx_hbm, table_hbm, out_hbm, idx_v, rows_v, obuf_v, gsem, osem):
    wid = lax.axis_index("s") * NC + lax.axis_index("c")
    base_chunk = wid * ch_per_w
    pltpu.sync_copy(idx_hbm.at[pl.ds(base_chunk, ch_per_w)], idx_v)

    def gather(c, p):
      return pltpu.make_async_copy(
          table_hbm.at[idx_v.at[c]], rows_v.at[p], gsem.at[p]
      )

    def store(c, q):
      ci = base_chunk + c
      f = ci // blocks_per_field
      b0 = (ci % blocks_per_field) * BCHUNK
      return pltpu.make_async_copy(
          obuf_v.at[q, :, pl.ds(0, BCHUNK)],
          out_hbm.at[f, :, pl.ds(b0, BCHUNK)],
          osem.at[q],
      )

    iota16 = lax.iota(jnp.int32, 16)
    jvecs = [iota16 + 16 * g for g in range(EMB_DIM // 16)]

    gather(0, 0).start()
    gather(1, 1).start()

    @pl.loop(0, ch_per_w, step=4)
    def _(c0):
      for p in range(4):
        c = c0 + p
        q = p % 2
        gather(c, p).wait()

        @pl.when(c + 2 < ch_per_w)
        def _():
          gather(c + 2, (p + 2) % 4).start()

        @pl.when(c >= 2)
        def _():
          store(c - 2, q).wait()

        src = rows_v.at[p]
        dst = obuf_v.at[q]

        @plsc.parallel_loop(0, BCHUNK, unroll=4)
        def _(b):
          bb = jnp.full((16,), b, jnp.int32)
          for g in range(EMB_DIM // 16):
            v = src[b, pl.ds(16 * g, 16)]
            plsc.store_scatter(dst, [jvecs[g], bb], v)

        store(c, q).start()

    store(ch_per_w - 2, 0).wait()
    store(ch_per_w - 1, 1).wait()

  return grab


def kernel(input, embedding_weight):
  b, f = input.shape
  idx = input.astype(jnp.int32).T.reshape(f * (b // BCHUNK), BCHUNK)
  grab = _make_gather(b, f, embedding_weight.shape[0])
  out_t = grab(idx, embedding_weight)
  return out_t.transpose(2, 0, 1).astype(jnp.bfloat16)
